# Initial kernel scaffold; baseline (speedup 1.0000x reference)
import jax
import jax.numpy as jnp
from jax.experimental import pallas as pl


def kernel(scores):
    raise NotImplementedError("write your pallas kernel here")


# speedup vs baseline: 1.0090x; 1.0090x over previous
import jax
import jax.numpy as jnp
from jax.experimental import pallas as pl

_N = 32768
_R = 256
_C = 128
_K = 64
_BIG = 1 << 30


def _body(x_ref, out_ref):
    x0 = x_ref[:]
    lin = (jax.lax.broadcasted_iota(jnp.int32, (_R, _C), 0) * _C
           + jax.lax.broadcasted_iota(jnp.int32, (_R, _C), 1))
    neginf = jnp.float32(-jnp.inf)
    jcol = jax.lax.broadcasted_iota(jnp.int32, (1, _C), 1)

    def step(i, carry):
        x, vals, idxs, cs, s = carry
        m = jnp.max(x)
        j = jnp.min(jnp.where(x == m, lin, jnp.int32(_BIG)))
        x = jnp.where(lin == j, neginf, x)
        s = s + m
        sel = jcol == i
        vals = jnp.where(sel, m, vals)
        idxs = jnp.where(sel, j, idxs)
        cs = jnp.where(sel, s, cs)
        return x, vals, idxs, cs, s

    _, vals, idxs, cs, _ = jax.lax.fori_loop(
        0, _K, step,
        (x0, jnp.zeros((1, _C), jnp.float32), jnp.zeros((1, _C), jnp.int32),
         jnp.zeros((1, _C), jnp.float32), jnp.float32(0.0)))

    kvec = (jcol + 1).astype(jnp.float32)
    support = ((vals - (cs - 1.0) / kvec) > 0.0) & (jcol < _K)
    kz = jnp.sum(support.astype(jnp.int32))
    cs_at = jnp.sum(jnp.where(jcol == kz - 1, cs, 0.0))
    tau = (cs_at - 1.0) / kz.astype(jnp.float32)

    row0 = x0[0:1, :]
    avail = row0 <= tau

    def fstep(t, carry):
        fill, cprev = carry
        cand = jnp.where(avail & (jcol > cprev), jcol, jnp.int32(_BIG))
        c = jnp.min(cand)
        sel = jcol == (kz + t)
        fill = jnp.where(sel, c, fill)
        return fill, c

    fill, _ = jax.lax.fori_loop(
        0, _K, fstep, (jnp.zeros((1, _C), jnp.int32), jnp.int32(-1)))

    out = jnp.where(jcol < kz, idxs, fill)
    out_ref[:] = jnp.broadcast_to(out, (8, _C))


def kernel(scores):
    x = scores.reshape(_R, _C)
    out = pl.pallas_call(
        _body,
        out_shape=jax.ShapeDtypeStruct((8, _C), jnp.int32),
    )(x)
    return out[0, :_K]
